# Initial kernel scaffold; baseline (speedup 1.0000x reference)
import jax
import jax.numpy as jnp
from jax.experimental import pallas as pl


def kernel(inputs):
    raise NotImplementedError("write your pallas kernel here")


# speedup vs baseline: 4.4289x; 4.4289x over previous
import jax
import jax.numpy as jnp
from jax.experimental import pallas as pl
from jax.experimental.pallas import tpu as pltpu

_SLAB = 128


def _body(x_hbm, o_ref, slab, fb, idx_smem, insem, fbsem, outsem):
    b, t, f = x_hbm.shape

    slab_copies = []
    for i in range(b):
        c = pltpu.make_async_copy(
            x_hbm.at[i, :, pl.ds(0, _SLAB)], slab.at[i], insem.at[i]
        )
        c.start()
        slab_copies.append(c)

    row_copies = []
    for i in range(b):
        slab_copies[i].wait()
        x = slab[i]
        m = jnp.max(jnp.abs(x), axis=1, keepdims=True)
        cnt = jnp.sum((m > 0.0).astype(jnp.int32))
        idx_smem[i] = jnp.maximum(cnt - 1, 0)

        @pl.when(cnt < t)
        def _():
            fc = pltpu.make_async_copy(x_hbm.at[i], fb, fbsem)
            fc.start()
            fc.wait()
            mf = jnp.max(jnp.abs(fb[...]), axis=1, keepdims=True)
            cf = jnp.sum((mf > 0.0).astype(jnp.int32))
            idx_smem[i] = jnp.maximum(cf - 1, 0)

        rc = pltpu.make_async_copy(
            x_hbm.at[i, pl.ds(idx_smem[i], 1), :],
            o_ref.at[pl.ds(i, 1)],
            outsem.at[i],
        )
        rc.start()
        row_copies.append(rc)

    for c in row_copies:
        c.wait()


def kernel(inputs):
    b, t, f = inputs.shape

    return pl.pallas_call(
        _body,
        in_specs=[pl.BlockSpec(memory_space=pl.ANY)],
        out_specs=pl.BlockSpec((b, f), lambda: (0, 0)),
        out_shape=jax.ShapeDtypeStruct((b, f), jnp.float32),
        scratch_shapes=[
            pltpu.VMEM((b, t, _SLAB), jnp.float32),
            pltpu.VMEM((t, f), jnp.float32),
            pltpu.SMEM((b,), jnp.int32),
            pltpu.SemaphoreType.DMA((b,)),
            pltpu.SemaphoreType.DMA,
            pltpu.SemaphoreType.DMA((b,)),
        ],
    )(inputs)
